# baseline (speedup 1.0000x reference)
import functools

import jax
import jax.numpy as jnp
from jax import lax
from jax.experimental import pallas as pl
from jax.experimental.pallas import tpu as pltpu
from jax.experimental.pallas import tpu_sc as plsc

NC, NS = 2, 16
NW = NC * NS
CHUNK = 128


@functools.partial(jax.jit, static_argnames=("n_per_w", "n_chunks", "dp"))
def _sc_gather(idx, table_p, *, n_per_w, n_chunks, dp):
    n = idx.shape[0]
    mesh = plsc.VectorSubcoreMesh(
        core_axis_name="c", subcore_axis_name="s", num_cores=NC,
        num_subcores=NS)

    @functools.partial(
        pl.kernel,
        out_type=jax.ShapeDtypeStruct((n, dp), jnp.float32),
        mesh=mesh,
        scratch_types=[
            pltpu.VMEM((CHUNK,), jnp.int32),
            pltpu.VMEM((CHUNK, dp), jnp.float32),
            pltpu.SemaphoreType.DMA,
        ],
        compiler_params=pltpu.CompilerParams(use_tc_tiling_on_sc=False),
    )
    def k(idx_hbm, table_hbm, out_hbm, idx_v, rows_v, sem):
        wid = lax.axis_index("s") * NC + lax.axis_index("c")
        base = wid * n_per_w

        def body(i, carry):
            off = base + i * CHUNK
            pltpu.sync_copy(idx_hbm.at[pl.ds(off, CHUNK)], idx_v)
            pltpu.async_copy(table_hbm.at[idx_v], rows_v, sem).wait()
            pltpu.sync_copy(rows_v, out_hbm.at[pl.ds(off, CHUNK)])
            return carry

        lax.fori_loop(0, n_chunks, body, 0)

    return k(idx, table_p)


def kernel(input, table):
    b, h = input.shape
    v, d = table.shape
    dp = (d + 7) // 8 * 8
    n = b * h
    assert n % (NW * CHUNK) == 0
    n_per_w = n // NW
    idx = input.reshape(n).astype(jnp.int32)
    table_p = jnp.pad(table, ((0, 0), (0, dp - d)))
    out = _sc_gather(idx, table_p, n_per_w=n_per_w,
                     n_chunks=n_per_w // CHUNK, dp=dp)
    return out[:, :d].reshape(b, h, d)


# speedup vs baseline: 1.3430x; 1.3430x over previous
import functools

import jax
import jax.numpy as jnp
from jax import lax
from jax.experimental import pallas as pl
from jax.experimental.pallas import tpu as pltpu
from jax.experimental.pallas import tpu_sc as plsc

NC, NS = 2, 16
NW = NC * NS
CHUNK = 128
LANE = 128


@functools.partial(jax.jit, static_argnames=("n_per_w", "n_chunks", "d"))
def _sc_gather(idx, table_seg, *, n_per_w, n_chunks, d):
    n = idx.shape[0]
    mesh = plsc.VectorSubcoreMesh(
        core_axis_name="c", subcore_axis_name="s", num_cores=NC,
        num_subcores=NS)

    @functools.partial(
        pl.kernel,
        out_type=jax.ShapeDtypeStruct((n, d), jnp.float32),
        mesh=mesh,
        scratch_types=[
            pltpu.VMEM((CHUNK,), jnp.int32),
            pltpu.VMEM((CHUNK,), jnp.int32),
            pltpu.VMEM((CHUNK,), jnp.int32),
            pltpu.VMEM((CHUNK,), jnp.int32),
            pltpu.VMEM((CHUNK, d), jnp.float32),
            pltpu.VMEM((CHUNK, LANE), jnp.float32),
            pltpu.SemaphoreType.DMA,
        ],
        compiler_params=pltpu.CompilerParams(use_tc_tiling_on_sc=True,
                                             needs_layout_passes=False),
    )
    def k(idx_hbm, table_hbm, out_hbm, idx_v, e0_v, e1_v, e2_v, rows_v,
          tail_v, sem):
        wid = lax.axis_index("s") * NC + lax.axis_index("c")
        base = wid * n_per_w
        ntile = d // LANE
        tail = d - ntile * LANE
        lanes = lax.iota(jnp.int32, 16)

        def body(i, carry):
            row0 = base + i * CHUNK
            pltpu.sync_copy(idx_hbm.at[pl.ds(row0, CHUNK)], idx_v)
            for t in range(CHUNK // 16):
                sl = pl.ds(16 * t, 16)
                e = idx_v[sl] * 3
                e0_v[sl] = e
                e1_v[sl] = e + 1
                e2_v[sl] = e + 2
            for s, e_v in enumerate((e0_v, e1_v)):
                pltpu.async_copy(table_hbm.at[e_v],
                                 rows_v.at[:, pl.ds(s * LANE, LANE)], sem)
            pltpu.async_copy(table_hbm.at[e2_v], tail_v, sem)
            for s, e_v in enumerate((e0_v, e1_v)):
                pltpu.make_async_copy(
                    table_hbm.at[e_v],
                    rows_v.at[:, pl.ds(s * LANE, LANE)], sem).wait()
            pltpu.make_async_copy(table_hbm.at[e2_v], tail_v, sem).wait()

            def tail_body(r, c2):
                rows_v[r, pl.ds(ntile * LANE, 16)] = tail_v[r, pl.ds(0, 16)]
                rows_v[r, pl.ds(ntile * LANE + 16, 16)] = \
                    tail_v[r, pl.ds(16, 16)]
                plsc.store_scatter(
                    rows_v,
                    [jnp.broadcast_to(r, (16,)),
                     ntile * LANE + 32 + lanes],
                    tail_v[r, pl.ds(32, 16)],
                    mask=lanes < tail - 32)
                return c2

            lax.fori_loop(0, CHUNK, tail_body, 0)
            pltpu.sync_copy(rows_v, out_hbm.at[pl.ds(row0, CHUNK)])
            return carry

        lax.fori_loop(0, n_chunks, body, 0)

    return k(idx, table_seg)


def kernel(input, table):
    b, h = input.shape
    v, d = table.shape
    nseg = (d + LANE - 1) // LANE
    dp = nseg * LANE
    n = b * h
    assert n % (NW * CHUNK) == 0
    n_per_w = n // NW
    idx = input.reshape(n).astype(jnp.int32)
    table_seg = jnp.pad(table, ((0, 0), (0, dp - d))).reshape(v * nseg, LANE)
    out = _sc_gather(idx, table_seg, n_per_w=n_per_w,
                     n_chunks=n_per_w // CHUNK, d=d)
    return out.reshape(b, h, d)


# speedup vs baseline: 1.4854x; 1.1060x over previous
import functools

import jax
import jax.numpy as jnp
from jax import lax
from jax.experimental import pallas as pl
from jax.experimental.pallas import tpu as pltpu
from jax.experimental.pallas import tpu_sc as plsc

NC, NS = 2, 16
NW = NC * NS
CHUNK = 80
LANE = 128
TAILW = 128


@functools.partial(jax.jit, static_argnames=("n_per_w", "n_chunks", "d"))
def _sc_gather(idx, table_main, *, n_per_w, n_chunks, d):
    n = idx.shape[0]
    mesh = plsc.VectorSubcoreMesh(
        core_axis_name="c", subcore_axis_name="s", num_cores=NC,
        num_subcores=NS)

    @functools.partial(
        pl.kernel,
        out_type=jax.ShapeDtypeStruct((n, d), jnp.float32),
        mesh=mesh,
        scratch_types=[
            [pltpu.VMEM((CHUNK,), jnp.int32) for _ in range(2)],
            [pltpu.VMEM((CHUNK,), jnp.int32) for _ in range(2)],
            [pltpu.VMEM((CHUNK,), jnp.int32) for _ in range(2)],
            [pltpu.VMEM((CHUNK,), jnp.int32) for _ in range(2)],
            [pltpu.VMEM((CHUNK, d), jnp.float32) for _ in range(2)],
            [pltpu.VMEM((CHUNK, TAILW), jnp.float32) for _ in range(2)],
            [pltpu.SemaphoreType.DMA for _ in range(2)],
            [pltpu.SemaphoreType.DMA for _ in range(2)],
            [pltpu.SemaphoreType.DMA for _ in range(2)],
        ],
        compiler_params=pltpu.CompilerParams(use_tc_tiling_on_sc=True,
                                             needs_layout_passes=False),
    )
    def k(idx_hbm, tmain_hbm, out_hbm, idx_v, e0_v, e1_v, e2_v, rows_v,
          tail_v, sem_i, sem_g, sem_w):
        wid = lax.axis_index("s") * NC + lax.axis_index("c")
        base = wid * n_per_w
        ntile = d // LANE
        tail = d - ntile * LANE
        lanes = lax.iota(jnp.int32, 16)
        tail_mask = lanes < tail - 32

        def fire_idx(i, b):
            off = base + i * CHUNK
            pltpu.async_copy(idx_hbm.at[pl.ds(off, CHUNK)], idx_v[b],
                             sem_i[b])

        def wait_idx(b):
            pltpu.make_async_copy(idx_hbm.at[pl.ds(0, CHUNK)], idx_v[b],
                                  sem_i[b]).wait()

        def exp_fire_gathers(b):
            for t in range(CHUNK // 16):
                sl = pl.ds(16 * t, 16)
                e = idx_v[b][sl] * 3
                e0_v[b][sl] = e
                e1_v[b][sl] = e + 1
                e2_v[b][sl] = e + 2
            pltpu.async_copy(tmain_hbm.at[e0_v[b]],
                             rows_v[b].at[:, pl.ds(0, LANE)], sem_g[b])
            pltpu.async_copy(tmain_hbm.at[e1_v[b]],
                             rows_v[b].at[:, pl.ds(LANE, LANE)], sem_g[b])
            pltpu.async_copy(tmain_hbm.at[e2_v[b]], tail_v[b], sem_g[b])

        def wait_gathers(b):
            pltpu.make_async_copy(tmain_hbm.at[e0_v[b]],
                                  rows_v[b].at[:, pl.ds(0, LANE)],
                                  sem_g[b]).wait()
            pltpu.make_async_copy(tmain_hbm.at[e1_v[b]],
                                  rows_v[b].at[:, pl.ds(LANE, LANE)],
                                  sem_g[b]).wait()
            pltpu.make_async_copy(tmain_hbm.at[e2_v[b]], tail_v[b],
                                  sem_g[b]).wait()

        def bridge(b):
            def tail_body(r, c):
                rows_v[b][r, pl.ds(ntile * LANE, 16)] = \
                    tail_v[b][r, pl.ds(0, 16)]
                rows_v[b][r, pl.ds(ntile * LANE + 16, 16)] = \
                    tail_v[b][r, pl.ds(16, 16)]
                plsc.store_scatter(
                    rows_v[b],
                    [jnp.broadcast_to(r, (16,)),
                     ntile * LANE + 32 + lanes],
                    tail_v[b][r, pl.ds(32, 16)],
                    mask=tail_mask)
                return c

            lax.fori_loop(0, CHUNK, tail_body, 0)

        def fire_write(i, b):
            off = base + i * CHUNK
            pltpu.async_copy(rows_v[b], out_hbm.at[pl.ds(off, CHUNK)],
                             sem_w[b])

        def wait_write(b):
            pltpu.make_async_copy(rows_v[b], out_hbm.at[pl.ds(0, CHUNK)],
                                  sem_w[b]).wait()

        def pair(p, first):
            for b in (0, 1):
                i = 2 * p + b
                nxt = lax.rem(i + 1, n_chunks)
                if not (first and b == 0):
                    wait_write(1 - b)
                fire_idx(nxt, 1 - b)
                wait_gathers(b)
                bridge(b)
                fire_write(i, b)
                wait_idx(1 - b)
                exp_fire_gathers(1 - b)

        fire_idx(0, 0)
        wait_idx(0)
        exp_fire_gathers(0)
        pair(0, True)
        lax.fori_loop(1, n_chunks // 2, lambda p, c: (pair(p, False), c)[1],
                      0)
        wait_gathers(0)
        wait_write(1)

    return k(idx, table_main)


def kernel(input, table):
    b, h = input.shape
    v, d = table.shape
    n = b * h
    assert n % (NW * CHUNK) == 0
    n_per_w = n // NW
    idx = input.reshape(n).astype(jnp.int32)
    nseg = (d + LANE - 1) // LANE
    table_main = jnp.pad(table, ((0, 0), (0, nseg * LANE - d))
                         ).reshape(v * nseg, LANE)
    out = _sc_gather(idx, table_main, n_per_w=n_per_w,
                     n_chunks=n_per_w // CHUNK, d=d)
    return out.reshape(b, h, d)


# speedup vs baseline: 1.6134x; 1.0862x over previous
import functools

import jax
import jax.numpy as jnp
from jax import lax
from jax.experimental import pallas as pl
from jax.experimental.pallas import tpu as pltpu
from jax.experimental.pallas import tpu_sc as plsc

NC, NS = 2, 16
NW = NC * NS
CHUNK = 80
LANE = 128
TAILW = 128


@functools.partial(jax.jit, static_argnames=("n_per_w", "n_chunks", "d"))
def _sc_gather(idx, table_main, *, n_per_w, n_chunks, d):
    n = idx.shape[0]
    mesh = plsc.VectorSubcoreMesh(
        core_axis_name="c", subcore_axis_name="s", num_cores=NC,
        num_subcores=NS)

    @functools.partial(
        pl.kernel,
        out_type=jax.ShapeDtypeStruct((n, d), jnp.float32),
        mesh=mesh,
        scratch_types=[
            [pltpu.VMEM((CHUNK,), jnp.int32) for _ in range(2)],
            [pltpu.VMEM((CHUNK,), jnp.int32) for _ in range(2)],
            [pltpu.VMEM((CHUNK,), jnp.int32) for _ in range(2)],
            [pltpu.VMEM((CHUNK,), jnp.int32) for _ in range(2)],
            [pltpu.VMEM((CHUNK, d), jnp.float32) for _ in range(2)],
            [pltpu.VMEM((CHUNK, TAILW), jnp.float32) for _ in range(2)],
            [pltpu.SemaphoreType.DMA for _ in range(2)],
            [pltpu.SemaphoreType.DMA for _ in range(2)],
            [pltpu.SemaphoreType.DMA for _ in range(2)],
        ],
        compiler_params=pltpu.CompilerParams(use_tc_tiling_on_sc=True,
                                             needs_layout_passes=False),
    )
    def k(idx_hbm, tmain_hbm, out_hbm, idx_v, e0_v, e1_v, e2_v, rows_v,
          tail_v, sem_i, sem_g, sem_w):
        wid = lax.axis_index("s") * NC + lax.axis_index("c")
        base = wid * n_per_w
        ntile = d // LANE
        tail = d - ntile * LANE
        lanes = lax.iota(jnp.int32, 16)
        tail_mask = lanes < tail - 32

        def fire_idx(i, b):
            off = base + i * CHUNK
            pltpu.async_copy(idx_hbm.at[pl.ds(off, CHUNK)], idx_v[b],
                             sem_i[b])

        def wait_idx(b):
            pltpu.make_async_copy(idx_hbm.at[pl.ds(0, CHUNK)], idx_v[b],
                                  sem_i[b]).wait()

        def exp_fire_gathers(b):
            for t in range(CHUNK // 16):
                sl = pl.ds(16 * t, 16)
                e = idx_v[b][sl] * 3
                e0_v[b][sl] = e
                e1_v[b][sl] = e + 1
                e2_v[b][sl] = e + 2
            pltpu.async_copy(tmain_hbm.at[e0_v[b]],
                             rows_v[b].at[:, pl.ds(0, LANE)], sem_g[b])
            pltpu.async_copy(tmain_hbm.at[e1_v[b]],
                             rows_v[b].at[:, pl.ds(LANE, LANE)], sem_g[b])
            pltpu.async_copy(tmain_hbm.at[e2_v[b]], tail_v[b], sem_g[b])

        def wait_gathers(b):
            pltpu.make_async_copy(tmain_hbm.at[e0_v[b]],
                                  rows_v[b].at[:, pl.ds(0, LANE)],
                                  sem_g[b]).wait()
            pltpu.make_async_copy(tmain_hbm.at[e1_v[b]],
                                  rows_v[b].at[:, pl.ds(LANE, LANE)],
                                  sem_g[b]).wait()
            pltpu.make_async_copy(tmain_hbm.at[e2_v[b]], tail_v[b],
                                  sem_g[b]).wait()

        def bridge(b):
            def tail_body(r, c):
                rows_v[b][r, pl.ds(ntile * LANE, 16)] = \
                    tail_v[b][r, pl.ds(0, 16)]
                rows_v[b][r, pl.ds(ntile * LANE + 16, 16)] = \
                    tail_v[b][r, pl.ds(16, 16)]
                plsc.store_scatter(
                    rows_v[b],
                    [jnp.broadcast_to(r, (16,)),
                     ntile * LANE + 32 + lanes],
                    tail_v[b][r, pl.ds(32, 16)],
                    mask=tail_mask)
                return c

            lax.fori_loop(0, CHUNK, tail_body, 0)

        def fire_write(i, b):
            off = base + i * CHUNK
            pltpu.async_copy(rows_v[b], out_hbm.at[pl.ds(off, CHUNK)],
                             sem_w[b])

        def wait_write(b):
            pltpu.make_async_copy(rows_v[b], out_hbm.at[pl.ds(0, CHUNK)],
                                  sem_w[b]).wait()

        def pair(p, first):
            for b in (0, 1):
                i = 2 * p + b
                nxt = lax.rem(i + 1, n_chunks)
                if not (first and b == 0):
                    wait_write(1 - b)
                fire_idx(nxt, 1 - b)
                wait_gathers(b)
                bridge(b)
                fire_write(i, b)
                wait_idx(1 - b)
                exp_fire_gathers(1 - b)

        fire_idx(0, 0)
        wait_idx(0)
        exp_fire_gathers(0)
        pair(0, True)
        lax.fori_loop(1, n_chunks // 2, lambda p, c: (pair(p, False), c)[1],
                      0)
        wait_gathers(0)
        wait_write(1)

    return k(idx, table_main)


def kernel(input, table):
    b, h = input.shape
    v, d = table.shape
    n = b * h
    assert n % (NW * CHUNK) == 0
    n_per_w = n // NW
    idx = input.reshape(n).astype(jnp.int32)
    nseg = (d + LANE - 1) // LANE
    table_main = jnp.stack(
        [table[:, 0 * LANE:1 * LANE], table[:, 1 * LANE:2 * LANE],
         jnp.pad(table[:, 2 * LANE:], ((0, 0), (0, nseg * LANE - d)))],
        axis=1).reshape(v * nseg, LANE)
    out = _sc_gather(idx, table_main, n_per_w=n_per_w,
                     n_chunks=n_per_w // CHUNK, d=d)
    return out.reshape(b, h, d)


# speedup vs baseline: 1.6222x; 1.0055x over previous
import functools

import jax
import jax.numpy as jnp
from jax import lax
from jax.experimental import pallas as pl
from jax.experimental.pallas import tpu as pltpu
from jax.experimental.pallas import tpu_sc as plsc

NC, NS = 2, 16
NW = NC * NS
CHUNK = 80
LANE = 128
TAILW = 128


@functools.partial(jax.jit, static_argnames=("n_per_w", "n_chunks", "d"))
def _sc_gather(idx, table_main, *, n_per_w, n_chunks, d):
    n = idx.shape[0]
    mesh = plsc.VectorSubcoreMesh(
        core_axis_name="c", subcore_axis_name="s", num_cores=NC,
        num_subcores=NS)

    @functools.partial(
        pl.kernel,
        out_type=jax.ShapeDtypeStruct((n, d), jnp.float32),
        mesh=mesh,
        scratch_types=[
            [pltpu.VMEM((CHUNK,), jnp.int32) for _ in range(2)],
            [pltpu.VMEM((CHUNK,), jnp.int32) for _ in range(2)],
            [pltpu.VMEM((CHUNK,), jnp.int32) for _ in range(2)],
            [pltpu.VMEM((CHUNK,), jnp.int32) for _ in range(2)],
            [pltpu.VMEM((CHUNK, d), jnp.float32) for _ in range(2)],
            [pltpu.VMEM((CHUNK, TAILW), jnp.float32) for _ in range(2)],
            [pltpu.SemaphoreType.DMA for _ in range(2)],
            [pltpu.SemaphoreType.DMA for _ in range(2)],
            [pltpu.SemaphoreType.DMA for _ in range(2)],
        ],
        compiler_params=pltpu.CompilerParams(use_tc_tiling_on_sc=True,
                                             needs_layout_passes=False),
    )
    def k(idx_hbm, tmain_hbm, out_hbm, idx_v, e0_v, e1_v, e2_v, rows_v,
          tail_v, sem_i, sem_g, sem_w):
        wid = lax.axis_index("s") * NC + lax.axis_index("c")
        base = wid * n_per_w
        ntile = d // LANE
        tail = d - ntile * LANE
        lanes = lax.iota(jnp.int32, 16)
        tail_mask = lanes < tail - 32

        def fire_idx(i, b):
            off = base + i * CHUNK
            pltpu.async_copy(idx_hbm.at[pl.ds(off, CHUNK)], idx_v[b],
                             sem_i[b])

        def wait_idx(b):
            pltpu.make_async_copy(idx_hbm.at[pl.ds(0, CHUNK)], idx_v[b],
                                  sem_i[b]).wait()

        def exp_fire_gathers(b):
            for t in range(CHUNK // 16):
                sl = pl.ds(16 * t, 16)
                e = idx_v[b][sl] * 3
                e0_v[b][sl] = e
                e1_v[b][sl] = e + 1
                e2_v[b][sl] = e + 2
            pltpu.async_copy(tmain_hbm.at[e0_v[b]],
                             rows_v[b].at[:, pl.ds(0, LANE)], sem_g[b])
            pltpu.async_copy(tmain_hbm.at[e1_v[b]],
                             rows_v[b].at[:, pl.ds(LANE, LANE)], sem_g[b])
            pltpu.async_copy(tmain_hbm.at[e2_v[b]], tail_v[b], sem_g[b])

        def wait_gathers(b):
            pltpu.make_async_copy(tmain_hbm.at[e0_v[b]],
                                  rows_v[b].at[:, pl.ds(0, LANE)],
                                  sem_g[b]).wait()
            pltpu.make_async_copy(tmain_hbm.at[e1_v[b]],
                                  rows_v[b].at[:, pl.ds(LANE, LANE)],
                                  sem_g[b]).wait()
            pltpu.make_async_copy(tmain_hbm.at[e2_v[b]], tail_v[b],
                                  sem_g[b]).wait()

        def bridge(b):
            unroll = 8

            def tail_body(rq, c):
                for u in range(unroll):
                    r = rq * unroll + u
                    rows_v[b][r, pl.ds(ntile * LANE, 16)] = \
                        tail_v[b][r, pl.ds(0, 16)]
                    rows_v[b][r, pl.ds(ntile * LANE + 16, 16)] = \
                        tail_v[b][r, pl.ds(16, 16)]
                    plsc.store_scatter(
                        rows_v[b],
                        [jnp.broadcast_to(r, (16,)),
                         ntile * LANE + 32 + lanes],
                        tail_v[b][r, pl.ds(32, 16)],
                        mask=tail_mask)
                return c

            lax.fori_loop(0, CHUNK // unroll, tail_body, 0)

        def fire_write(i, b):
            off = base + i * CHUNK
            pltpu.async_copy(rows_v[b], out_hbm.at[pl.ds(off, CHUNK)],
                             sem_w[b])

        def wait_write(b):
            pltpu.make_async_copy(rows_v[b], out_hbm.at[pl.ds(0, CHUNK)],
                                  sem_w[b]).wait()

        def pair(p, first):
            for b in (0, 1):
                i = 2 * p + b
                nxt = lax.rem(i + 1, n_chunks)
                if not (first and b == 0):
                    wait_write(1 - b)
                fire_idx(nxt, 1 - b)
                wait_gathers(b)
                bridge(b)
                fire_write(i, b)
                wait_idx(1 - b)
                exp_fire_gathers(1 - b)

        fire_idx(0, 0)
        wait_idx(0)
        exp_fire_gathers(0)
        pair(0, True)
        lax.fori_loop(1, n_chunks // 2, lambda p, c: (pair(p, False), c)[1],
                      0)
        wait_gathers(0)
        wait_write(1)

    return k(idx, table_main)


def kernel(input, table):
    b, h = input.shape
    v, d = table.shape
    n = b * h
    assert n % (NW * CHUNK) == 0
    n_per_w = n // NW
    idx = input.reshape(n).astype(jnp.int32)
    nseg = (d + LANE - 1) // LANE
    table_main = jnp.stack(
        [table[:, 0 * LANE:1 * LANE], table[:, 1 * LANE:2 * LANE],
         jnp.pad(table[:, 2 * LANE:], ((0, 0), (0, nseg * LANE - d)))],
        axis=1).reshape(v * nseg, LANE)
    out = _sc_gather(idx, table_main, n_per_w=n_per_w,
                     n_chunks=n_per_w // CHUNK, d=d)
    return out.reshape(b, h, d)


# speedup vs baseline: 1.7745x; 1.0939x over previous
import functools

import jax
import jax.numpy as jnp
from jax import lax
from jax.experimental import pallas as pl
from jax.experimental.pallas import tpu as pltpu
from jax.experimental.pallas import tpu_sc as plsc

NC, NS = 2, 16
NW = NC * NS
CHUNK = 80
LANE = 128
TAILW = 128


@functools.partial(jax.jit, static_argnames=("n_per_w", "n_chunks", "d", "v"))
def _sc_gather(idx, table_main, *, n_per_w, n_chunks, d, v):
    n = idx.shape[0]
    mesh = plsc.VectorSubcoreMesh(
        core_axis_name="c", subcore_axis_name="s", num_cores=NC,
        num_subcores=NS)

    @functools.partial(
        pl.kernel,
        out_type=jax.ShapeDtypeStruct((n, d), jnp.float32),
        mesh=mesh,
        scratch_types=[
            [pltpu.VMEM((CHUNK,), jnp.int32) for _ in range(2)],
            [pltpu.VMEM((CHUNK,), jnp.int32) for _ in range(2)],
            [pltpu.VMEM((CHUNK,), jnp.int32) for _ in range(2)],
            [pltpu.VMEM((CHUNK,), jnp.int32) for _ in range(2)],
            [pltpu.VMEM((CHUNK, d), jnp.float32) for _ in range(2)],
            [pltpu.VMEM((CHUNK, TAILW), jnp.float32) for _ in range(2)],
            [pltpu.SemaphoreType.DMA for _ in range(2)],
            [pltpu.SemaphoreType.DMA for _ in range(2)],
            [pltpu.SemaphoreType.DMA for _ in range(2)],
        ],
        compiler_params=pltpu.CompilerParams(use_tc_tiling_on_sc=True,
                                             needs_layout_passes=False),
    )
    def k(idx_hbm, tmain_hbm, out_hbm, idx_v, e0_v, e1_v, e2_v, rows_v,
          tail_v, sem_i, sem_g, sem_w):
        wid = lax.axis_index("s") * NC + lax.axis_index("c")
        base = wid * n_per_w
        ntile = d // LANE
        tail = d - ntile * LANE
        lanes = lax.iota(jnp.int32, 16)
        tail_mask = lanes < tail - 32

        def fire_idx(i, b):
            off = base + i * CHUNK
            pltpu.async_copy(idx_hbm.at[pl.ds(off, CHUNK)], idx_v[b],
                             sem_i[b])

        def wait_idx(b):
            pltpu.make_async_copy(idx_hbm.at[pl.ds(0, CHUNK)], idx_v[b],
                                  sem_i[b]).wait()

        def exp_fire_gathers(b):
            for t in range(CHUNK // 16):
                sl = pl.ds(16 * t, 16)
                e = idx_v[b][sl]
                e0_v[b][sl] = e
                e1_v[b][sl] = e + v
                e2_v[b][sl] = e + 2 * v
            pltpu.async_copy(tmain_hbm.at[e0_v[b]],
                             rows_v[b].at[:, pl.ds(0, LANE)], sem_g[b])
            pltpu.async_copy(tmain_hbm.at[e1_v[b]],
                             rows_v[b].at[:, pl.ds(LANE, LANE)], sem_g[b])
            pltpu.async_copy(tmain_hbm.at[e2_v[b]], tail_v[b], sem_g[b])

        def wait_gathers(b):
            pltpu.make_async_copy(tmain_hbm.at[e0_v[b]],
                                  rows_v[b].at[:, pl.ds(0, LANE)],
                                  sem_g[b]).wait()
            pltpu.make_async_copy(tmain_hbm.at[e1_v[b]],
                                  rows_v[b].at[:, pl.ds(LANE, LANE)],
                                  sem_g[b]).wait()
            pltpu.make_async_copy(tmain_hbm.at[e2_v[b]], tail_v[b],
                                  sem_g[b]).wait()

        def bridge(b):
            unroll = 8

            def tail_body(rq, c):
                for u in range(unroll):
                    r = rq * unroll + u
                    rows_v[b][r, pl.ds(ntile * LANE, 16)] = \
                        tail_v[b][r, pl.ds(0, 16)]
                    rows_v[b][r, pl.ds(ntile * LANE + 16, 16)] = \
                        tail_v[b][r, pl.ds(16, 16)]
                    plsc.store_scatter(
                        rows_v[b],
                        [jnp.broadcast_to(r, (16,)),
                         ntile * LANE + 32 + lanes],
                        tail_v[b][r, pl.ds(32, 16)],
                        mask=tail_mask)
                return c

            lax.fori_loop(0, CHUNK // unroll, tail_body, 0)

        def fire_write(i, b):
            off = base + i * CHUNK
            pltpu.async_copy(rows_v[b], out_hbm.at[pl.ds(off, CHUNK)],
                             sem_w[b])

        def wait_write(b):
            pltpu.make_async_copy(rows_v[b], out_hbm.at[pl.ds(0, CHUNK)],
                                  sem_w[b]).wait()

        def pair(p, first):
            for b in (0, 1):
                i = 2 * p + b
                nxt = lax.rem(i + 1, n_chunks)
                if not (first and b == 0):
                    wait_write(1 - b)
                fire_idx(nxt, 1 - b)
                wait_gathers(b)
                bridge(b)
                fire_write(i, b)
                wait_idx(1 - b)
                exp_fire_gathers(1 - b)

        fire_idx(0, 0)
        wait_idx(0)
        exp_fire_gathers(0)
        pair(0, True)
        lax.fori_loop(1, n_chunks // 2, lambda p, c: (pair(p, False), c)[1],
                      0)
        wait_gathers(0)
        wait_write(1)

    return k(idx, table_main)


def kernel(input, table):
    b, h = input.shape
    v, d = table.shape
    n = b * h
    assert n % (NW * CHUNK) == 0
    n_per_w = n // NW
    idx = input.reshape(n).astype(jnp.int32)
    nseg = (d + LANE - 1) // LANE
    table_main = jnp.concatenate(
        [table[:, 0 * LANE:1 * LANE], table[:, 1 * LANE:2 * LANE],
         jnp.pad(table[:, 2 * LANE:], ((0, 0), (0, nseg * LANE - d)))],
        axis=0)
    out = _sc_gather(idx, table_main, n_per_w=n_per_w,
                     n_chunks=n_per_w // CHUNK, d=d, v=v)
    return out.reshape(b, h, d)
